# baseline (speedup 1.0000x reference)
import functools

import jax
import jax.numpy as jnp
from jax import lax
from jax.experimental import pallas as pl
from jax.experimental.pallas import tpu as pltpu
from jax.experimental.pallas import tpu_sc as plsc

_N = 10000
_E = 320000
_G = 64

_B = 640
_IC = 128


def _make_agg(F):
    S = F // 8
    J = max(1, S // 32)
    P = max(1, 32 // S)
    epp = _E // P
    nb = epp // _B
    mesh = plsc.VectorSubcoreMesh(core_axis_name="c", subcore_axis_name="s")

    @functools.partial(
        pl.kernel,
        out_type=jax.ShapeDtypeStruct((P, _N, F), jnp.float32),
        mesh=mesh,
        compiler_params=pltpu.CompilerParams(use_tc_tiling_on_sc=False,
                                             needs_layout_passes=False),
        scratch_types=[
            pltpu.VMEM((_B,), jnp.int32),
            pltpu.VMEM((_B,), jnp.int32),
            pltpu.VMEM((_B,), jnp.int32),
            pltpu.VMEM((_B, 8), jnp.float32),
            pltpu.VMEM((_N, 8), jnp.float32),
            pltpu.SemaphoreType.DMA,
        ],
    )
    def agg_kernel(x8, srcv, dstv, out, src_v, dst_v, idx_v, rows_v, acc_v,
                   sem):
        wid = lax.axis_index("s") * 2 + lax.axis_index("c")
        vlane = lax.iota(jnp.int32, 16)
        hi8 = vlane >> 3
        fidx = vlane & 7
        part = wid % P
        zeros16 = jnp.zeros((16,), jnp.float32)

        for t in range(J):
            sl = (wid // P) * J + t

            def zbody(j, carry):
                plsc.store_scatter(acc_v, [2 * j + hi8, fidx], zeros16)
                return carry
            lax.fori_loop(0, _N // 2, zbody, 0, unroll=8)

            def bbody(b, carry):
                e0 = part * epp + b * _B
                pltpu.sync_copy(srcv.at[pl.ds(e0, _B)], src_v)
                pltpu.sync_copy(dstv.at[pl.ds(e0, _B)], dst_v)

                def ibody(j, c):
                    idx_v[pl.ds(j * 16, 16)] = (
                        src_v[pl.ds(j * 16, 16)] * S + sl)
                    return c
                lax.fori_loop(0, _B // 16, ibody, 0, unroll=8)

                copies = [
                    pltpu.async_copy(
                        x8.at[idx_v.at[pl.ds(k * _IC, _IC)]],
                        rows_v.at[pl.ds(k * _IC, _IC), :], sem)
                    for k in range(_B // _IC)
                ]
                for c in copies:
                    c.wait()

                def pbody(j, c):
                    didx = 2 * j + hi8
                    d01 = plsc.load_gather(dst_v, [didx])
                    r16 = plsc.load_gather(rows_v, [didx, fidx])
                    plsc.addupdate_scatter(acc_v, [d01, fidx], r16)
                    return c
                lax.fori_loop(0, _B // 2, pbody, 0, unroll=4)
                return carry
            lax.fori_loop(0, nb, bbody, 0)

            pltpu.sync_copy(acc_v, out.at[part, :, pl.ds(8 * sl, 8)])

    return agg_kernel


_agg_kernels = {F: _make_agg(F) for F in (128, 256, 512)}


def _agg(h, src, dst):
    n, f = h.shape
    parts = _agg_kernels[f](h.reshape(-1, 8), src, dst)
    return parts


def _dense_body(nparts, parts_ref, x_ref, wr_ref, wx_ref, b_ref, o_ref):
    agg = parts_ref[0]
    for p in range(1, nparts):
        agg = agg + parts_ref[p]
    acc = jnp.dot(agg, wr_ref[...], preferred_element_type=jnp.float32)
    acc = acc + jnp.dot(x_ref[...], wx_ref[...], preferred_element_type=jnp.float32)
    o_ref[...] = jnp.maximum(acc + b_ref[...], 0.0)


def _dense(parts, x, w_rel, b, w_root):
    n, f = x.shape
    nparts = parts.shape[0]
    o = w_rel.shape[1]
    bn = 400
    return pl.pallas_call(
        functools.partial(_dense_body, nparts),
        grid=(n // bn,),
        in_specs=[
            pl.BlockSpec((nparts, bn, f), lambda i: (0, i, 0)),
            pl.BlockSpec((bn, f), lambda i: (i, 0)),
            pl.BlockSpec((f, o), lambda i: (0, 0)),
            pl.BlockSpec((f, o), lambda i: (0, 0)),
            pl.BlockSpec((1, o), lambda i: (0, 0)),
        ],
        out_specs=pl.BlockSpec((bn, o), lambda i: (i, 0)),
        out_shape=jax.ShapeDtypeStruct((n, o), jnp.float32),
    )(parts, x, w_rel, w_root, b.reshape(1, -1))


def _head_body(h_ref, batch_ref, wm1_ref, bm1_ref, wm2_ref, bm2_ref,
               wm3_ref, bm3_ref, o_ref, pooled_ref, cnt_ref):
    i = pl.program_id(0)
    nsteps = pl.num_programs(0)

    @pl.when(i == 0)
    def _init():
        pooled_ref[...] = jnp.zeros_like(pooled_ref)
        cnt_ref[...] = jnp.zeros_like(cnt_ref)

    bids = batch_ref[0, 0, :]
    gids = jax.lax.broadcasted_iota(jnp.int32, (_G, bids.shape[0]), 0)
    onehot = (gids == bids[None, :]).astype(jnp.float32)
    pooled_ref[...] += jnp.dot(onehot, h_ref[...],
                               preferred_element_type=jnp.float32)
    cnt_ref[...] += jnp.sum(onehot, axis=1, keepdims=True)

    @pl.when(i == nsteps - 1)
    def _final():
        cnt = jnp.maximum(cnt_ref[...], 1.0)
        h = pooled_ref[...] / cnt
        h = jnp.maximum(jnp.dot(h, wm1_ref[...],
                                preferred_element_type=jnp.float32)
                        + bm1_ref[...], 0.0)
        h = jnp.maximum(jnp.dot(h, wm2_ref[...],
                                preferred_element_type=jnp.float32)
                        + bm2_ref[...], 0.0)
        logits = jnp.dot(h, wm3_ref[...],
                         preferred_element_type=jnp.float32) + bm3_ref[...]
        m = jnp.max(logits, axis=-1, keepdims=True)
        z = logits - m
        lse = jnp.log(jnp.sum(jnp.exp(z), axis=-1, keepdims=True))
        o_ref[...] = z - lse


def _head(h, batch, wm1, bm1, wm2, bm2, wm3, bm3):
    n, f = h.shape
    bn = 400
    nsteps = n // bn
    batch3 = batch.reshape(nsteps, 1, bn)
    c = wm3.shape[1]
    h1 = wm1.shape[1]
    h2 = wm2.shape[1]
    return pl.pallas_call(
        _head_body,
        grid=(nsteps,),
        in_specs=[
            pl.BlockSpec((bn, f), lambda i: (i, 0)),
            pl.BlockSpec((1, 1, bn), lambda i: (i, 0, 0)),
            pl.BlockSpec((f, h1), lambda i: (0, 0)),
            pl.BlockSpec((1, h1), lambda i: (0, 0)),
            pl.BlockSpec((h1, h2), lambda i: (0, 0)),
            pl.BlockSpec((1, h2), lambda i: (0, 0)),
            pl.BlockSpec((h2, c), lambda i: (0, 0)),
            pl.BlockSpec((1, c), lambda i: (0, 0)),
        ],
        out_specs=pl.BlockSpec((_G, c), lambda i: (0, 0)),
        out_shape=jax.ShapeDtypeStruct((_G, c), jnp.float32),
        scratch_shapes=[
            pltpu.VMEM((_G, f), jnp.float32),
            pltpu.VMEM((_G, 1), jnp.float32),
        ],
    )(h, batch3, wm1, bm1.reshape(1, -1), wm2, bm2.reshape(1, -1),
      wm3, bm3.reshape(1, -1))


def kernel(x, edge_index, batch,
           W1_rel, b1, W1_root,
           W2_rel, b2, W2_root,
           W3_rel, b3, W3_root,
           Wm1, bm1, Wm2, bm2, Wm3, bm3):
    src = edge_index[0]
    dst = edge_index[1]

    h = _dense(_agg(x, src, dst), x, W1_rel, b1, W1_root)
    h = _dense(_agg(h, src, dst), h, W2_rel, b2, W2_root)
    h = _dense(_agg(h, src, dst), h, W3_rel, b3, W3_root)
    return _head(h, batch, Wm1, bm1, Wm2, bm2, Wm3, bm3)


# speedup vs baseline: 1.6053x; 1.6053x over previous
import functools

import jax
import jax.numpy as jnp
from jax import lax
from jax.experimental import pallas as pl
from jax.experimental.pallas import tpu as pltpu
from jax.experimental.pallas import tpu_sc as plsc

_N = 10000
_E = 320000
_G = 64

_IC = 128


def _make_agg(F):
    S = F // 8
    J = max(1, S // 32)
    P = max(1, 32 // S)
    epp = _E // P
    B = 640 if P == 2 else 1280
    nb = epp // B
    nc = B // _IC
    mesh = plsc.VectorSubcoreMesh(core_axis_name="c", subcore_axis_name="s")

    @functools.partial(
        pl.kernel,
        out_type=jax.ShapeDtypeStruct((P, _N, F), jnp.float32),
        mesh=mesh,
        compiler_params=pltpu.CompilerParams(use_tc_tiling_on_sc=False,
                                             needs_layout_passes=False),
        scratch_types=[
            [pltpu.VMEM((B,), jnp.int32)] * 2,
            [pltpu.VMEM((B,), jnp.int32)] * 2,
            [pltpu.VMEM((B,), jnp.int32)] * 2,
            [pltpu.VMEM((B, 8), jnp.float32)] * 2,
            pltpu.VMEM((_N, 8), jnp.float32),
            [pltpu.SemaphoreType.DMA] * 2,
            [pltpu.SemaphoreType.DMA] * 2,
        ],
    )
    def agg_kernel(x8, srcv, dstv, out, src_v, dst_v, idx_v, rows_v, acc_v,
                   sem_sd, sem_g):
        wid = lax.axis_index("s") * 2 + lax.axis_index("c")
        vlane = lax.iota(jnp.int32, 16)
        hi8 = vlane >> 3
        fidx = vlane & 7
        part = wid % P
        ebase = part * epp
        zeros16 = jnp.zeros((16,), jnp.float32)
        nbc = jnp.int32(nb - 1)

        def fire_sd(s, b):
            e0 = ebase + lax.min(b, nbc) * B
            pltpu.make_async_copy(srcv.at[pl.ds(e0, B)], src_v[s],
                                  sem_sd[s]).start()
            pltpu.make_async_copy(dstv.at[pl.ds(e0, B)], dst_v[s],
                                  sem_sd[s]).start()

        def wait_sd(s):
            pltpu.make_async_copy(srcv.at[pl.ds(0, B)], src_v[s],
                                  sem_sd[s]).wait()
            pltpu.make_async_copy(dstv.at[pl.ds(0, B)], dst_v[s],
                                  sem_sd[s]).wait()

        def gather_descs(s):
            return [pltpu.make_async_copy(
                x8.at[idx_v[s].at[pl.ds(k * _IC, _IC)]],
                rows_v[s].at[pl.ds(k * _IC, _IC), :], sem_g[s])
                for k in range(nc)]

        def idx_and_fire(s, sl):
            def ibody(j, c):
                idx_v[s][pl.ds(j * 16, 16)] = (
                    src_v[s][pl.ds(j * 16, 16)] * S + sl)
                return c
            lax.fori_loop(0, B // 16, ibody, 0, unroll=8)
            for d in gather_descs(s):
                d.start()

        def wait_g(s):
            for d in gather_descs(s):
                d.wait()

        def pairs(s):
            def pbody(j, c):
                didx = 2 * j + hi8
                d01 = plsc.load_gather(dst_v[s], [didx])
                r16 = plsc.load_gather(rows_v[s], [didx, fidx])
                plsc.addupdate_scatter(acc_v, [d01, fidx], r16)
                return c
            lax.fori_loop(0, B // 2, pbody, 0, unroll=8)

        for t in range(J):
            sl = (wid // P) * J + t

            def zbody(j, carry):
                plsc.store_scatter(acc_v, [2 * j + hi8, fidx], zeros16)
                return carry
            lax.fori_loop(0, _N // 2, zbody, 0, unroll=8)

            fire_sd(0, jnp.int32(0))
            fire_sd(1, jnp.int32(1))
            wait_sd(0)
            idx_and_fire(0, sl)

            def dbody(i, carry):
                b0 = 2 * i
                wait_sd(1)
                idx_and_fire(1, sl)
                wait_g(0)
                pairs(0)
                fire_sd(0, b0 + 2)
                wait_g(1)
                pairs(1)
                fire_sd(1, b0 + 3)
                wait_sd(0)
                idx_and_fire(0, sl)
                return carry
            lax.fori_loop(0, nb // 2, dbody, 0)

            wait_g(0)
            wait_sd(1)

            pltpu.sync_copy(acc_v, out.at[part, :, pl.ds(8 * sl, 8)])

    return agg_kernel


_agg_kernels = {F: _make_agg(F) for F in (128, 256, 512)}


def _agg(h, src, dst):
    n, f = h.shape
    parts = _agg_kernels[f](h.reshape(-1, 8), src, dst)
    return parts


def _dense_body(nparts, parts_ref, x_ref, wr_ref, wx_ref, b_ref, o_ref):
    agg = parts_ref[0]
    for p in range(1, nparts):
        agg = agg + parts_ref[p]
    acc = jnp.dot(agg, wr_ref[...], preferred_element_type=jnp.float32)
    acc = acc + jnp.dot(x_ref[...], wx_ref[...], preferred_element_type=jnp.float32)
    o_ref[...] = jnp.maximum(acc + b_ref[...], 0.0)


def _dense(parts, x, w_rel, b, w_root):
    n, f = x.shape
    nparts = parts.shape[0]
    o = w_rel.shape[1]
    bn = 400
    return pl.pallas_call(
        functools.partial(_dense_body, nparts),
        grid=(n // bn,),
        in_specs=[
            pl.BlockSpec((nparts, bn, f), lambda i: (0, i, 0)),
            pl.BlockSpec((bn, f), lambda i: (i, 0)),
            pl.BlockSpec((f, o), lambda i: (0, 0)),
            pl.BlockSpec((f, o), lambda i: (0, 0)),
            pl.BlockSpec((1, o), lambda i: (0, 0)),
        ],
        out_specs=pl.BlockSpec((bn, o), lambda i: (i, 0)),
        out_shape=jax.ShapeDtypeStruct((n, o), jnp.float32),
    )(parts, x, w_rel, w_root, b.reshape(1, -1))


def _head_body(h_ref, batch_ref, wm1_ref, bm1_ref, wm2_ref, bm2_ref,
               wm3_ref, bm3_ref, o_ref, pooled_ref, cnt_ref):
    i = pl.program_id(0)
    nsteps = pl.num_programs(0)

    @pl.when(i == 0)
    def _init():
        pooled_ref[...] = jnp.zeros_like(pooled_ref)
        cnt_ref[...] = jnp.zeros_like(cnt_ref)

    bids = batch_ref[0, 0, :]
    gids = jax.lax.broadcasted_iota(jnp.int32, (_G, bids.shape[0]), 0)
    onehot = (gids == bids[None, :]).astype(jnp.float32)
    pooled_ref[...] += jnp.dot(onehot, h_ref[...],
                               preferred_element_type=jnp.float32)
    cnt_ref[...] += jnp.sum(onehot, axis=1, keepdims=True)

    @pl.when(i == nsteps - 1)
    def _final():
        cnt = jnp.maximum(cnt_ref[...], 1.0)
        h = pooled_ref[...] / cnt
        h = jnp.maximum(jnp.dot(h, wm1_ref[...],
                                preferred_element_type=jnp.float32)
                        + bm1_ref[...], 0.0)
        h = jnp.maximum(jnp.dot(h, wm2_ref[...],
                                preferred_element_type=jnp.float32)
                        + bm2_ref[...], 0.0)
        logits = jnp.dot(h, wm3_ref[...],
                         preferred_element_type=jnp.float32) + bm3_ref[...]
        m = jnp.max(logits, axis=-1, keepdims=True)
        z = logits - m
        lse = jnp.log(jnp.sum(jnp.exp(z), axis=-1, keepdims=True))
        o_ref[...] = z - lse


def _head(h, batch, wm1, bm1, wm2, bm2, wm3, bm3):
    n, f = h.shape
    bn = 400
    nsteps = n // bn
    batch3 = batch.reshape(nsteps, 1, bn)
    c = wm3.shape[1]
    h1 = wm1.shape[1]
    h2 = wm2.shape[1]
    return pl.pallas_call(
        _head_body,
        grid=(nsteps,),
        in_specs=[
            pl.BlockSpec((bn, f), lambda i: (i, 0)),
            pl.BlockSpec((1, 1, bn), lambda i: (i, 0, 0)),
            pl.BlockSpec((f, h1), lambda i: (0, 0)),
            pl.BlockSpec((1, h1), lambda i: (0, 0)),
            pl.BlockSpec((h1, h2), lambda i: (0, 0)),
            pl.BlockSpec((1, h2), lambda i: (0, 0)),
            pl.BlockSpec((h2, c), lambda i: (0, 0)),
            pl.BlockSpec((1, c), lambda i: (0, 0)),
        ],
        out_specs=pl.BlockSpec((_G, c), lambda i: (0, 0)),
        out_shape=jax.ShapeDtypeStruct((_G, c), jnp.float32),
        scratch_shapes=[
            pltpu.VMEM((_G, f), jnp.float32),
            pltpu.VMEM((_G, 1), jnp.float32),
        ],
    )(h, batch3, wm1, bm1.reshape(1, -1), wm2, bm2.reshape(1, -1),
      wm3, bm3.reshape(1, -1))


def kernel(x, edge_index, batch,
           W1_rel, b1, W1_root,
           W2_rel, b2, W2_root,
           W3_rel, b3, W3_root,
           Wm1, bm1, Wm2, bm2, Wm3, bm3):
    src = edge_index[0]
    dst = edge_index[1]

    h = _dense(_agg(x, src, dst), x, W1_rel, b1, W1_root)
    h = _dense(_agg(h, src, dst), h, W2_rel, b2, W2_root)
    h = _dense(_agg(h, src, dst), h, W3_rel, b3, W3_root)
    return _head(h, batch, Wm1, bm1, Wm2, bm2, Wm3, bm3)


# speedup vs baseline: 8.9850x; 5.5969x over previous
import functools

import jax
import jax.numpy as jnp
from jax import lax
from jax.experimental import pallas as pl
from jax.experimental.pallas import tpu as pltpu
from jax.experimental.pallas import tpu_sc as plsc

_N = 10000
_E = 320000
_G = 64

_D = 128


def _maybe_when(cond, fn):
    if isinstance(cond, bool):
        if cond:
            fn()
    else:
        pl.when(cond)(fn)


def _make_agg(F):
    S = F // _D
    npass = max(1, S // 2)
    P = 2 if S == 1 else 1
    ept = _E // (16 * P)
    _B = 80 if P == 2 else 160
    _IC = _B // 5
    _NC = 5
    nb = ept // _B
    npt = _N // 16
    mesh = plsc.VectorSubcoreMesh(core_axis_name="c", subcore_axis_name="s")

    @functools.partial(
        pl.kernel,
        out_type=jax.ShapeDtypeStruct((P, _N, F), jnp.float32),
        mesh=mesh,
        compiler_params=pltpu.CompilerParams(use_tc_tiling_on_sc=False,
                                             needs_layout_passes=False),
        scratch_types=[
            [pltpu.VMEM((_B,), jnp.int32)] * 2,
            [pltpu.VMEM((_B,), jnp.int32)] * 2,
            [pltpu.VMEM((_B,), jnp.int32)] * 2,
            [[pltpu.VMEM((_IC,), jnp.int32)] * _NC] * 2,
            [pltpu.VMEM((_B, _D), jnp.float32)] * 2,
            pltpu.VMEM_SHARED((_N, _D), jnp.float32),
            [pltpu.SemaphoreType.DMA] * 2,
            [pltpu.SemaphoreType.DMA] * 2,
            [pltpu.SemaphoreType.DMA] * 2,
        ],
    )
    def agg_kernel(x128, srcv, dstv, out, src_st, dst_st, gidx, sidx, rows,
                   acc_sh, sem_sd, sem_g, sem_a):
        sc = lax.axis_index("c")
        tid = lax.axis_index("s")
        ebase0 = (sc * (_E // 2) if P == 2 else 0) + tid * ept
        part = sc if P == 2 else 0

        def fire_sd(s, b):
            e0 = ebase0 + b * _B
            pltpu.make_async_copy(srcv.at[pl.ds(e0, _B)], src_st[s],
                                  sem_sd[s]).start()
            pltpu.make_async_copy(dstv.at[pl.ds(e0, _B)], dst_st[s],
                                  sem_sd[s]).start()

        def wait_sd(s):
            pltpu.make_async_copy(srcv.at[pl.ds(0, _B)], src_st[s],
                                  sem_sd[s]).wait()
            pltpu.make_async_copy(dstv.at[pl.ds(0, _B)], dst_st[s],
                                  sem_sd[s]).wait()

        def idx_compute(s, chunk):
            for q in range(_B // 16):
                slq = pl.ds(q * 16, 16)
                if S > 1:
                    gidx[s][slq] = src_st[s][slq] * S + chunk
                else:
                    gidx[s][slq] = src_st[s][slq]
                sidx[s][q // (_IC // 16)][
                    pl.ds((q % (_IC // 16)) * 16, 16)] = dst_st[s][slq]

        def fire_gathers(s):
            for k in range(_NC):
                pltpu.make_async_copy(
                    x128.at[gidx[s].at[pl.ds(k * _IC, _IC)]],
                    rows[s].at[pl.ds(k * _IC, _IC), :], sem_g[s]).start()

        def wait_gathers(s):
            pltpu.make_async_copy(x128.at[pl.ds(0, _B), :], rows[s],
                                  sem_g[s]).wait()

        def fire_scatter(s):
            for k in range(_NC):
                pltpu.async_copy(rows[s].at[pl.ds(k * _IC, _IC), :],
                                 acc_sh.at[sidx[s][k]], sem_a[s], add=True)

        def wait_scatter(s):
            pltpu.make_async_copy(x128.at[pl.ds(0, _B), :], rows[s],
                                  sem_a[s]).wait()

        def step(b, j, chunk):
            o = 1 - j
            wait_gathers(j)
            fire_scatter(j)

            def stage_next():
                wait_sd(o)
                idx_compute(o, chunk)

                def drain_prev():
                    wait_scatter(o)
                _maybe_when(b >= 1, drain_prev)
                fire_gathers(o)
            _maybe_when(b + 1 < nb, stage_next)

            def prefetch():
                fire_sd(j, b + 2)
            _maybe_when(b + 2 < nb, prefetch)

        zrows = rows[0]

        def zero_pass():
            z16 = jnp.zeros((16,), jnp.float32)

            def zb(i, c):
                zrows[i // (_D // 16), pl.ds((i % (_D // 16)) * 16, 16)] = z16
                return c
            lax.fori_loop(0, _B * (_D // 16), zb, 0, unroll=8)
            base = tid * npt
            for off in range(0, npt, _B):
                w = min(_B, npt - off)
                pltpu.sync_copy(zrows.at[pl.ds(0, w), :],
                                acc_sh.at[pl.ds(base + off, w)])

        for p in range(npass):
            chunk = sc * npass + p if S > 1 else 0

            zero_pass()
            plsc.subcore_barrier()

            fire_sd(0, jnp.int32(0))
            fire_sd(1, jnp.int32(1))
            wait_sd(0)
            idx_compute(0, chunk)
            fire_gathers(0)

            def group(g, carry):
                step(2 * g, 0, chunk)
                step(2 * g + 1, 1, chunk)
                return carry
            lax.fori_loop(0, nb // 2, group, 0)
            if nb % 2:
                step(nb - 1, (nb - 1) % 2, chunk)

            wait_scatter(0)
            wait_scatter(1)
            plsc.subcore_barrier()

            base = tid * npt
            pltpu.sync_copy(
                acc_sh.at[pl.ds(base, npt)],
                out.at[part, pl.ds(base, npt), pl.ds(chunk * _D, _D)])
            plsc.subcore_barrier()

    return agg_kernel


_agg_kernels = {F: _make_agg(F) for F in (128, 256, 512)}


def _agg(h, src, dst):
    n, f = h.shape
    parts = _agg_kernels[f](h.reshape(-1, _D), src, dst)
    return parts


def _dense_body(nparts, parts_ref, x_ref, wr_ref, wx_ref, b_ref, o_ref):
    agg = parts_ref[0]
    for p in range(1, nparts):
        agg = agg + parts_ref[p]
    acc = jnp.dot(agg, wr_ref[...], preferred_element_type=jnp.float32)
    acc = acc + jnp.dot(x_ref[...], wx_ref[...], preferred_element_type=jnp.float32)
    o_ref[...] = jnp.maximum(acc + b_ref[...], 0.0)


def _dense(parts, x, w_rel, b, w_root):
    n, f = x.shape
    nparts = parts.shape[0]
    o = w_rel.shape[1]
    bn = 400
    return pl.pallas_call(
        functools.partial(_dense_body, nparts),
        grid=(n // bn,),
        in_specs=[
            pl.BlockSpec((nparts, bn, f), lambda i: (0, i, 0)),
            pl.BlockSpec((bn, f), lambda i: (i, 0)),
            pl.BlockSpec((f, o), lambda i: (0, 0)),
            pl.BlockSpec((f, o), lambda i: (0, 0)),
            pl.BlockSpec((1, o), lambda i: (0, 0)),
        ],
        out_specs=pl.BlockSpec((bn, o), lambda i: (i, 0)),
        out_shape=jax.ShapeDtypeStruct((n, o), jnp.float32),
    )(parts, x, w_rel, w_root, b.reshape(1, -1))


def _head_body(h_ref, batch_ref, wm1_ref, bm1_ref, wm2_ref, bm2_ref,
               wm3_ref, bm3_ref, o_ref, pooled_ref, cnt_ref):
    i = pl.program_id(0)
    nsteps = pl.num_programs(0)

    @pl.when(i == 0)
    def _init():
        pooled_ref[...] = jnp.zeros_like(pooled_ref)
        cnt_ref[...] = jnp.zeros_like(cnt_ref)

    bids = batch_ref[0, 0, :]
    gids = jax.lax.broadcasted_iota(jnp.int32, (_G, bids.shape[0]), 0)
    onehot = (gids == bids[None, :]).astype(jnp.float32)
    pooled_ref[...] += jnp.dot(onehot, h_ref[...],
                               preferred_element_type=jnp.float32)
    cnt_ref[...] += jnp.sum(onehot, axis=1, keepdims=True)

    @pl.when(i == nsteps - 1)
    def _final():
        cnt = jnp.maximum(cnt_ref[...], 1.0)
        h = pooled_ref[...] / cnt
        h = jnp.maximum(jnp.dot(h, wm1_ref[...],
                                preferred_element_type=jnp.float32)
                        + bm1_ref[...], 0.0)
        h = jnp.maximum(jnp.dot(h, wm2_ref[...],
                                preferred_element_type=jnp.float32)
                        + bm2_ref[...], 0.0)
        logits = jnp.dot(h, wm3_ref[...],
                         preferred_element_type=jnp.float32) + bm3_ref[...]
        m = jnp.max(logits, axis=-1, keepdims=True)
        z = logits - m
        lse = jnp.log(jnp.sum(jnp.exp(z), axis=-1, keepdims=True))
        o_ref[...] = z - lse


def _head(h, batch, wm1, bm1, wm2, bm2, wm3, bm3):
    n, f = h.shape
    bn = 400
    nsteps = n // bn
    batch3 = batch.reshape(nsteps, 1, bn)
    c = wm3.shape[1]
    h1 = wm1.shape[1]
    h2 = wm2.shape[1]
    return pl.pallas_call(
        _head_body,
        grid=(nsteps,),
        in_specs=[
            pl.BlockSpec((bn, f), lambda i: (i, 0)),
            pl.BlockSpec((1, 1, bn), lambda i: (i, 0, 0)),
            pl.BlockSpec((f, h1), lambda i: (0, 0)),
            pl.BlockSpec((1, h1), lambda i: (0, 0)),
            pl.BlockSpec((h1, h2), lambda i: (0, 0)),
            pl.BlockSpec((1, h2), lambda i: (0, 0)),
            pl.BlockSpec((h2, c), lambda i: (0, 0)),
            pl.BlockSpec((1, c), lambda i: (0, 0)),
        ],
        out_specs=pl.BlockSpec((_G, c), lambda i: (0, 0)),
        out_shape=jax.ShapeDtypeStruct((_G, c), jnp.float32),
        scratch_shapes=[
            pltpu.VMEM((_G, f), jnp.float32),
            pltpu.VMEM((_G, 1), jnp.float32),
        ],
    )(h, batch3, wm1, bm1.reshape(1, -1), wm2, bm2.reshape(1, -1),
      wm3, bm3.reshape(1, -1))


def kernel(x, edge_index, batch,
           W1_rel, b1, W1_root,
           W2_rel, b2, W2_root,
           W3_rel, b3, W3_root,
           Wm1, bm1, Wm2, bm2, Wm3, bm3):
    src = edge_index[0]
    dst = edge_index[1]

    h = _dense(_agg(x, src, dst), x, W1_rel, b1, W1_root)
    h = _dense(_agg(h, src, dst), h, W2_rel, b2, W2_root)
    h = _dense(_agg(h, src, dst), h, W3_rel, b3, W3_root)
    return _head(h, batch, Wm1, bm1, Wm2, bm2, Wm3, bm3)


# speedup vs baseline: 9.2436x; 1.0288x over previous
import functools

import jax
import jax.numpy as jnp
from jax import lax
from jax.experimental import pallas as pl
from jax.experimental.pallas import tpu as pltpu
from jax.experimental.pallas import tpu_sc as plsc

_N = 10000
_E = 320000
_G = 64

_D = 128


def _maybe_when(cond, fn):
    if isinstance(cond, bool):
        if cond:
            fn()
    else:
        pl.when(cond)(fn)


def _make_agg(F):
    S = F // _D
    npass = max(1, S // 2)
    P = 2 if S == 1 else 1
    ept = _E // (16 * P)
    _B = 80 if P == 2 else 160
    _IC = _B // 5
    _NC = 5
    nb = ept // _B
    npt = _N // 16
    mesh = plsc.VectorSubcoreMesh(core_axis_name="c", subcore_axis_name="s")

    @functools.partial(
        pl.kernel,
        out_type=jax.ShapeDtypeStruct((P, _N, F), jnp.float32),
        mesh=mesh,
        compiler_params=pltpu.CompilerParams(use_tc_tiling_on_sc=False,
                                             needs_layout_passes=False),
        scratch_types=[
            [pltpu.VMEM((_B,), jnp.int32)] * 2,
            [pltpu.VMEM((_B,), jnp.int32)] * 2,
            [pltpu.VMEM((_B,), jnp.int32)] * 2,
            [[pltpu.VMEM((_IC,), jnp.int32)] * _NC] * 2,
            [pltpu.VMEM((_B, _D), jnp.float32)] * 2,
            pltpu.VMEM_SHARED((_N, _D), jnp.float32),
            [pltpu.SemaphoreType.DMA] * 2,
            [pltpu.SemaphoreType.DMA] * 2,
            [pltpu.SemaphoreType.DMA] * 2,
        ],
    )
    def agg_kernel(x128, srcv, dstv, out, src_st, dst_st, gidx, sidx, rows,
                   acc_sh, sem_sd, sem_g, sem_a):
        sc = lax.axis_index("c")
        tid = lax.axis_index("s")
        ebase0 = (sc * (_E // 2) if P == 2 else 0) + tid * ept
        part = sc if P == 2 else 0

        def fire_sd(s, b):
            e0 = ebase0 + b * _B
            pltpu.make_async_copy(srcv.at[pl.ds(e0, _B)], src_st[s],
                                  sem_sd[s]).start()
            pltpu.make_async_copy(dstv.at[pl.ds(e0, _B)], dst_st[s],
                                  sem_sd[s]).start()

        def wait_sd(s):
            pltpu.make_async_copy(srcv.at[pl.ds(0, _B)], src_st[s],
                                  sem_sd[s]).wait()
            pltpu.make_async_copy(dstv.at[pl.ds(0, _B)], dst_st[s],
                                  sem_sd[s]).wait()

        def idx_compute(s, chunk):
            for q in range(_B // 16):
                slq = pl.ds(q * 16, 16)
                if S > 1:
                    gidx[s][slq] = src_st[s][slq] * S + chunk
                else:
                    gidx[s][slq] = src_st[s][slq]
                sidx[s][q // (_IC // 16)][
                    pl.ds((q % (_IC // 16)) * 16, 16)] = dst_st[s][slq]

        def fire_gathers(s):
            for k in range(_NC):
                pltpu.make_async_copy(
                    x128.at[gidx[s].at[pl.ds(k * _IC, _IC)]],
                    rows[s].at[pl.ds(k * _IC, _IC), :], sem_g[s]).start()

        def wait_gathers(s):
            pltpu.make_async_copy(x128.at[pl.ds(0, _B), :], rows[s],
                                  sem_g[s]).wait()

        def fire_scatter(s):
            for k in range(_NC):
                pltpu.async_copy(rows[s].at[pl.ds(k * _IC, _IC), :],
                                 acc_sh.at[sidx[s][k]], sem_a[s], add=True)

        def wait_scatter(s):
            pltpu.make_async_copy(x128.at[pl.ds(0, _B), :], rows[s],
                                  sem_a[s]).wait()

        def step(b, j, chunk):
            o = 1 - j
            wait_gathers(j)
            fire_scatter(j)

            def stage_next():
                wait_sd(o)
                idx_compute(o, chunk)

                def drain_prev():
                    wait_scatter(o)
                _maybe_when(b >= 1, drain_prev)
                fire_gathers(o)
            _maybe_when(b + 1 < nb, stage_next)

            def prefetch():
                fire_sd(j, b + 2)
            _maybe_when(b + 2 < nb, prefetch)

        zrows = rows[0]

        def zero_pass():
            z16 = jnp.zeros((16,), jnp.float32)

            def zb(i, c):
                zrows[i // (_D // 16), pl.ds((i % (_D // 16)) * 16, 16)] = z16
                return c
            lax.fori_loop(0, _B * (_D // 16), zb, 0, unroll=8)
            base = tid * npt
            for off in range(0, npt, _B):
                w = min(_B, npt - off)
                pltpu.sync_copy(zrows.at[pl.ds(0, w), :],
                                acc_sh.at[pl.ds(base + off, w)])

        for p in range(npass):
            chunk = sc * npass + p if S > 1 else 0

            zero_pass()
            plsc.subcore_barrier()

            fire_sd(0, jnp.int32(0))
            fire_sd(1, jnp.int32(1))
            wait_sd(0)
            idx_compute(0, chunk)
            fire_gathers(0)

            def group(g, carry):
                step(2 * g, 0, chunk)
                step(2 * g + 1, 1, chunk)
                return carry
            lax.fori_loop(0, nb // 2, group, 0)
            if nb % 2:
                step(nb - 1, (nb - 1) % 2, chunk)

            wait_scatter(0)
            wait_scatter(1)
            plsc.subcore_barrier()

            base = tid * npt
            pltpu.sync_copy(
                acc_sh.at[pl.ds(base, npt)],
                out.at[part, pl.ds(base, npt), pl.ds(chunk * _D, _D)])
            plsc.subcore_barrier()

    return agg_kernel


_agg_kernels = {F: _make_agg(F) for F in (128, 256, 512)}


def _agg(h, src, dst):
    n, f = h.shape
    parts = _agg_kernels[f](h.reshape(-1, _D), src, dst)
    return parts


def _dense_body(nparts, parts_ref, x_ref, wr_ref, wx_ref, b_ref, o_ref):
    agg = parts_ref[0]
    for p in range(1, nparts):
        agg = agg + parts_ref[p]
    acc = jnp.dot(agg, wr_ref[...], preferred_element_type=jnp.float32)
    acc = acc + jnp.dot(x_ref[...], wx_ref[...], preferred_element_type=jnp.float32)
    o_ref[...] = jnp.maximum(acc + b_ref[...], 0.0)


def _dense(parts, x, w_rel, b, w_root):
    n, f = x.shape
    nparts = parts.shape[0]
    o = w_rel.shape[1]
    bn = 400
    return pl.pallas_call(
        functools.partial(_dense_body, nparts),
        grid=(n // bn,),
        in_specs=[
            pl.BlockSpec((nparts, bn, f), lambda i: (0, i, 0)),
            pl.BlockSpec((bn, f), lambda i: (i, 0)),
            pl.BlockSpec((f, o), lambda i: (0, 0)),
            pl.BlockSpec((f, o), lambda i: (0, 0)),
            pl.BlockSpec((1, o), lambda i: (0, 0)),
        ],
        out_specs=pl.BlockSpec((bn, o), lambda i: (i, 0)),
        out_shape=jax.ShapeDtypeStruct((n, o), jnp.float32),
    )(parts, x, w_rel, w_root, b.reshape(1, -1))


def _head_body(parts_ref, x_ref, wr_ref, wx_ref, b_ref,
               batch_ref, wm1_ref, bm1_ref, wm2_ref, bm2_ref,
               wm3_ref, bm3_ref, o_ref, pooled_ref, cnt_ref):
    i = pl.program_id(0)
    nsteps = pl.num_programs(0)

    @pl.when(i == 0)
    def _init():
        pooled_ref[...] = jnp.zeros_like(pooled_ref)
        cnt_ref[...] = jnp.zeros_like(cnt_ref)

    acc = jnp.dot(parts_ref[0], wr_ref[...], preferred_element_type=jnp.float32)
    acc = acc + jnp.dot(x_ref[...], wx_ref[...],
                        preferred_element_type=jnp.float32)
    hblk = jnp.maximum(acc + b_ref[...], 0.0)

    bids = batch_ref[0, 0, :]
    gids = jax.lax.broadcasted_iota(jnp.int32, (_G, bids.shape[0]), 0)
    onehot = (gids == bids[None, :]).astype(jnp.float32)
    pooled_ref[...] += jnp.dot(onehot, hblk,
                               preferred_element_type=jnp.float32)
    cnt_ref[...] += jnp.sum(onehot, axis=1, keepdims=True)

    @pl.when(i == nsteps - 1)
    def _final():
        cnt = jnp.maximum(cnt_ref[...], 1.0)
        h = pooled_ref[...] / cnt
        h = jnp.maximum(jnp.dot(h, wm1_ref[...],
                                preferred_element_type=jnp.float32)
                        + bm1_ref[...], 0.0)
        h = jnp.maximum(jnp.dot(h, wm2_ref[...],
                                preferred_element_type=jnp.float32)
                        + bm2_ref[...], 0.0)
        logits = jnp.dot(h, wm3_ref[...],
                         preferred_element_type=jnp.float32) + bm3_ref[...]
        m = jnp.max(logits, axis=-1, keepdims=True)
        z = logits - m
        lse = jnp.log(jnp.sum(jnp.exp(z), axis=-1, keepdims=True))
        o_ref[...] = z - lse


def _head(parts, x, w_rel, b, w_root, batch, wm1, bm1, wm2, bm2, wm3, bm3):
    n, f = x.shape
    o = w_rel.shape[1]
    bn = 400
    nsteps = n // bn
    batch3 = batch.reshape(nsteps, 1, bn)
    c = wm3.shape[1]
    h1 = wm1.shape[1]
    h2 = wm2.shape[1]
    return pl.pallas_call(
        _head_body,
        grid=(nsteps,),
        in_specs=[
            pl.BlockSpec((1, bn, f), lambda i: (0, i, 0)),
            pl.BlockSpec((bn, f), lambda i: (i, 0)),
            pl.BlockSpec((f, o), lambda i: (0, 0)),
            pl.BlockSpec((f, o), lambda i: (0, 0)),
            pl.BlockSpec((1, o), lambda i: (0, 0)),
            pl.BlockSpec((1, 1, bn), lambda i: (i, 0, 0)),
            pl.BlockSpec((o, h1), lambda i: (0, 0)),
            pl.BlockSpec((1, h1), lambda i: (0, 0)),
            pl.BlockSpec((h1, h2), lambda i: (0, 0)),
            pl.BlockSpec((1, h2), lambda i: (0, 0)),
            pl.BlockSpec((h2, c), lambda i: (0, 0)),
            pl.BlockSpec((1, c), lambda i: (0, 0)),
        ],
        out_specs=pl.BlockSpec((_G, c), lambda i: (0, 0)),
        out_shape=jax.ShapeDtypeStruct((_G, c), jnp.float32),
        scratch_shapes=[
            pltpu.VMEM((_G, o), jnp.float32),
            pltpu.VMEM((_G, 1), jnp.float32),
        ],
    )(parts, x, w_rel, w_root, b.reshape(1, -1), batch3,
      wm1, bm1.reshape(1, -1), wm2, bm2.reshape(1, -1),
      wm3, bm3.reshape(1, -1))


def kernel(x, edge_index, batch,
           W1_rel, b1, W1_root,
           W2_rel, b2, W2_root,
           W3_rel, b3, W3_root,
           Wm1, bm1, Wm2, bm2, Wm3, bm3):
    src = edge_index[0]
    dst = edge_index[1]

    h = _dense(_agg(x, src, dst), x, W1_rel, b1, W1_root)
    h = _dense(_agg(h, src, dst), h, W2_rel, b2, W2_root)
    return _head(_agg(h, src, dst), h, W3_rel, b3, W3_root, batch,
                 Wm1, bm1, Wm2, bm2, Wm3, bm3)


# speedup vs baseline: 9.4075x; 1.0177x over previous
import functools

import jax
import jax.numpy as jnp
from jax import lax
from jax.experimental import pallas as pl
from jax.experimental.pallas import tpu as pltpu
from jax.experimental.pallas import tpu_sc as plsc

_N = 10000
_E = 320000
_G = 64

_D = 128


def _maybe_when(cond, fn):
    if isinstance(cond, bool):
        if cond:
            fn()
    else:
        pl.when(cond)(fn)


def _make_agg(F):
    S = F // _D
    npass = max(1, S // 2)
    P = 2 if S == 1 else 1
    ept = _E // (16 * P)
    _B = 80 if P == 2 else 160
    _IC = _B // 5
    _NC = 5
    nb = ept // _B
    npt = _N // 16
    mesh = plsc.VectorSubcoreMesh(core_axis_name="c", subcore_axis_name="s")

    @functools.partial(
        pl.kernel,
        out_type=jax.ShapeDtypeStruct((P, _N, F), jnp.float32),
        mesh=mesh,
        compiler_params=pltpu.CompilerParams(use_tc_tiling_on_sc=False,
                                             needs_layout_passes=False),
        scratch_types=[
            [pltpu.VMEM((_B,), jnp.int32)] * 2,
            [pltpu.VMEM((_B,), jnp.int32)] * 2,
            [pltpu.VMEM((_B,), jnp.int32)] * 2,
            [[pltpu.VMEM((_IC,), jnp.int32)] * _NC] * 2,
            [pltpu.VMEM((_B, _D), jnp.float32)] * 2,
            pltpu.VMEM_SHARED((_N, _D), jnp.float32),
            [pltpu.SemaphoreType.DMA] * 2,
            [pltpu.SemaphoreType.DMA] * 2,
            [pltpu.SemaphoreType.DMA] * 2,
        ],
    )
    def agg_kernel(x128, srcv, dstv, out, src_st, dst_st, gidx, sidx, rows,
                   acc_sh, sem_sd, sem_g, sem_a):
        sc = lax.axis_index("c")
        tid = lax.axis_index("s")
        ebase0 = (sc * (_E // 2) if P == 2 else 0) + tid * ept
        part = sc if P == 2 else 0

        def fire_sd(s, b):
            e0 = ebase0 + b * _B
            pltpu.make_async_copy(srcv.at[pl.ds(e0, _B)], src_st[s],
                                  sem_sd[s]).start()
            pltpu.make_async_copy(dstv.at[pl.ds(e0, _B)], dst_st[s],
                                  sem_sd[s]).start()

        def wait_sd(s):
            pltpu.make_async_copy(srcv.at[pl.ds(0, _B)], src_st[s],
                                  sem_sd[s]).wait()
            pltpu.make_async_copy(dstv.at[pl.ds(0, _B)], dst_st[s],
                                  sem_sd[s]).wait()

        def idx_compute(s, chunk):
            for q in range(_B // 16):
                slq = pl.ds(q * 16, 16)
                if S > 1:
                    gidx[s][slq] = src_st[s][slq] * S + chunk
                else:
                    gidx[s][slq] = src_st[s][slq]
                sidx[s][q // (_IC // 16)][
                    pl.ds((q % (_IC // 16)) * 16, 16)] = dst_st[s][slq]

        def fire_gathers(s):
            for k in range(_NC):
                pltpu.make_async_copy(
                    x128.at[gidx[s].at[pl.ds(k * _IC, _IC)]],
                    rows[s].at[pl.ds(k * _IC, _IC), :], sem_g[s]).start()

        def wait_gathers(s):
            pltpu.make_async_copy(x128.at[pl.ds(0, _B), :], rows[s],
                                  sem_g[s]).wait()

        def fire_scatter(s):
            for k in range(_NC):
                pltpu.async_copy(rows[s].at[pl.ds(k * _IC, _IC), :],
                                 acc_sh.at[sidx[s][k]], sem_a[s], add=True)

        def wait_scatter(s):
            pltpu.make_async_copy(x128.at[pl.ds(0, _B), :], rows[s],
                                  sem_a[s]).wait()

        def step(b, j, chunk):
            o = 1 - j
            wait_gathers(j)
            fire_scatter(j)

            def stage_next():
                wait_sd(o)
                idx_compute(o, chunk)

                def drain_prev():
                    wait_scatter(o)
                _maybe_when(b >= 1, drain_prev)
                fire_gathers(o)
            _maybe_when(b + 1 < nb, stage_next)

            def prefetch():
                fire_sd(j, b + 2)
            _maybe_when(b + 2 < nb, prefetch)

        zrows = rows[0]

        def zero_pass():
            z16 = jnp.zeros((16,), jnp.float32)

            def zb(i, c):
                zrows[i // (_D // 16), pl.ds((i % (_D // 16)) * 16, 16)] = z16
                return c
            lax.fori_loop(0, _B * (_D // 16), zb, 0, unroll=8)
            base = tid * npt
            for off in range(0, npt, _B):
                w = min(_B, npt - off)
                pltpu.sync_copy(zrows.at[pl.ds(0, w), :],
                                acc_sh.at[pl.ds(base + off, w)])

        for p in range(npass):
            chunk = sc * npass + p if S > 1 else 0

            zero_pass()
            plsc.subcore_barrier()

            fire_sd(0, jnp.int32(0))
            fire_sd(1, jnp.int32(1))
            wait_sd(0)
            idx_compute(0, chunk)
            fire_gathers(0)

            def group(g, carry):
                step(2 * g, 0, chunk)
                step(2 * g + 1, 1, chunk)
                return carry
            lax.fori_loop(0, nb // 2, group, 0)
            if nb % 2:
                step(nb - 1, (nb - 1) % 2, chunk)

            wait_scatter(0)
            wait_scatter(1)
            plsc.subcore_barrier()

            base = tid * npt
            pltpu.sync_copy(
                acc_sh.at[pl.ds(base, npt)],
                out.at[part, pl.ds(base, npt), pl.ds(chunk * _D, _D)])
            plsc.subcore_barrier()

    return agg_kernel


_agg_kernels = {F: _make_agg(F) for F in (128, 256, 512)}


def _agg(h, src, dst):
    n, f = h.shape
    parts = _agg_kernels[f](h.reshape(-1, _D), src, dst)
    return parts


def _dense_body(nparts, parts_ref, x_ref, wr_ref, wx_ref, b_ref, o_ref):
    agg = parts_ref[0]
    for p in range(1, nparts):
        agg = agg + parts_ref[p]
    acc = jnp.dot(agg, wr_ref[...], preferred_element_type=jnp.float32)
    acc = acc + jnp.dot(x_ref[...], wx_ref[...], preferred_element_type=jnp.float32)
    o_ref[...] = jnp.maximum(acc + b_ref[...], 0.0)


def _dense(parts, x, w_rel, b, w_root):
    n, f = x.shape
    nparts = parts.shape[0]
    o = w_rel.shape[1]
    bn = 2000
    return pl.pallas_call(
        functools.partial(_dense_body, nparts),
        grid=(n // bn,),
        in_specs=[
            pl.BlockSpec((nparts, bn, f), lambda i: (0, i, 0)),
            pl.BlockSpec((bn, f), lambda i: (i, 0)),
            pl.BlockSpec((f, o), lambda i: (0, 0)),
            pl.BlockSpec((f, o), lambda i: (0, 0)),
            pl.BlockSpec((1, o), lambda i: (0, 0)),
        ],
        out_specs=pl.BlockSpec((bn, o), lambda i: (i, 0)),
        out_shape=jax.ShapeDtypeStruct((n, o), jnp.float32),
    )(parts, x, w_rel, w_root, b.reshape(1, -1))


def _head_body(parts_ref, x_ref, wr_ref, wx_ref, b_ref,
               batch_ref, wm1_ref, bm1_ref, wm2_ref, bm2_ref,
               wm3_ref, bm3_ref, o_ref, pooled_ref, cnt_ref):
    i = pl.program_id(0)
    nsteps = pl.num_programs(0)

    @pl.when(i == 0)
    def _init():
        pooled_ref[...] = jnp.zeros_like(pooled_ref)
        cnt_ref[...] = jnp.zeros_like(cnt_ref)

    acc = jnp.dot(parts_ref[0], wr_ref[...], preferred_element_type=jnp.float32)
    acc = acc + jnp.dot(x_ref[...], wx_ref[...],
                        preferred_element_type=jnp.float32)
    hblk = jnp.maximum(acc + b_ref[...], 0.0)

    bids = batch_ref[0, 0, :]
    gids = jax.lax.broadcasted_iota(jnp.int32, (_G, bids.shape[0]), 0)
    onehot = (gids == bids[None, :]).astype(jnp.float32)
    pooled_ref[...] += jnp.dot(onehot, hblk,
                               preferred_element_type=jnp.float32)
    cnt_ref[...] += jnp.sum(onehot, axis=1, keepdims=True)

    @pl.when(i == nsteps - 1)
    def _final():
        cnt = jnp.maximum(cnt_ref[...], 1.0)
        h = pooled_ref[...] / cnt
        h = jnp.maximum(jnp.dot(h, wm1_ref[...],
                                preferred_element_type=jnp.float32)
                        + bm1_ref[...], 0.0)
        h = jnp.maximum(jnp.dot(h, wm2_ref[...],
                                preferred_element_type=jnp.float32)
                        + bm2_ref[...], 0.0)
        logits = jnp.dot(h, wm3_ref[...],
                         preferred_element_type=jnp.float32) + bm3_ref[...]
        m = jnp.max(logits, axis=-1, keepdims=True)
        z = logits - m
        lse = jnp.log(jnp.sum(jnp.exp(z), axis=-1, keepdims=True))
        o_ref[...] = z - lse


def _head(parts, x, w_rel, b, w_root, batch, wm1, bm1, wm2, bm2, wm3, bm3):
    n, f = x.shape
    o = w_rel.shape[1]
    bn = 400
    nsteps = n // bn
    batch3 = batch.reshape(nsteps, 1, bn)
    c = wm3.shape[1]
    h1 = wm1.shape[1]
    h2 = wm2.shape[1]
    return pl.pallas_call(
        _head_body,
        grid=(nsteps,),
        in_specs=[
            pl.BlockSpec((1, bn, f), lambda i: (0, i, 0)),
            pl.BlockSpec((bn, f), lambda i: (i, 0)),
            pl.BlockSpec((f, o), lambda i: (0, 0)),
            pl.BlockSpec((f, o), lambda i: (0, 0)),
            pl.BlockSpec((1, o), lambda i: (0, 0)),
            pl.BlockSpec((1, 1, bn), lambda i: (i, 0, 0)),
            pl.BlockSpec((o, h1), lambda i: (0, 0)),
            pl.BlockSpec((1, h1), lambda i: (0, 0)),
            pl.BlockSpec((h1, h2), lambda i: (0, 0)),
            pl.BlockSpec((1, h2), lambda i: (0, 0)),
            pl.BlockSpec((h2, c), lambda i: (0, 0)),
            pl.BlockSpec((1, c), lambda i: (0, 0)),
        ],
        out_specs=pl.BlockSpec((_G, c), lambda i: (0, 0)),
        out_shape=jax.ShapeDtypeStruct((_G, c), jnp.float32),
        scratch_shapes=[
            pltpu.VMEM((_G, o), jnp.float32),
            pltpu.VMEM((_G, 1), jnp.float32),
        ],
    )(parts, x, w_rel, w_root, b.reshape(1, -1), batch3,
      wm1, bm1.reshape(1, -1), wm2, bm2.reshape(1, -1),
      wm3, bm3.reshape(1, -1))


def kernel(x, edge_index, batch,
           W1_rel, b1, W1_root,
           W2_rel, b2, W2_root,
           W3_rel, b3, W3_root,
           Wm1, bm1, Wm2, bm2, Wm3, bm3):
    src = edge_index[0]
    dst = edge_index[1]

    h = _dense(_agg(x, src, dst), x, W1_rel, b1, W1_root)
    h = _dense(_agg(h, src, dst), h, W2_rel, b2, W2_root)
    return _head(_agg(h, src, dst), h, W3_rel, b3, W3_root, batch,
                 Wm1, bm1, Wm2, bm2, Wm3, bm3)


# speedup vs baseline: 9.5501x; 1.0152x over previous
import functools

import jax
import jax.numpy as jnp
from jax import lax
from jax.experimental import pallas as pl
from jax.experimental.pallas import tpu as pltpu
from jax.experimental.pallas import tpu_sc as plsc

_N = 10000
_E = 320000
_G = 64

_D = 128


def _maybe_when(cond, fn):
    if isinstance(cond, bool):
        if cond:
            fn()
    else:
        pl.when(cond)(fn)


def _make_agg(F):
    S = F // _D
    npass = max(1, S // 2)
    P = 2 if S == 1 else 1
    ept = _E // (16 * P)
    _B = 80 if P == 2 else 160
    _IC = _B // 5
    _NC = 5
    nb = ept // _B
    npt = _N // 16
    mesh = plsc.VectorSubcoreMesh(core_axis_name="c", subcore_axis_name="s")

    @functools.partial(
        pl.kernel,
        out_type=jax.ShapeDtypeStruct((P, _N, F), jnp.float32),
        mesh=mesh,
        compiler_params=pltpu.CompilerParams(use_tc_tiling_on_sc=False,
                                             needs_layout_passes=False),
        scratch_types=[
            [pltpu.VMEM((_B,), jnp.int32)] * 2,
            [pltpu.VMEM((_B,), jnp.int32)] * 2,
            [pltpu.VMEM((_B,), jnp.int32)] * 2,
            [[pltpu.VMEM((_IC,), jnp.int32)] * _NC] * 2,
            [pltpu.VMEM((_B, _D), jnp.float32)] * 2,
            pltpu.VMEM_SHARED((_N, _D), jnp.float32),
            [pltpu.SemaphoreType.DMA] * 2,
            [pltpu.SemaphoreType.DMA] * 2,
            [pltpu.SemaphoreType.DMA] * 2,
        ],
    )
    def agg_kernel(x128, srcv, dstv, out, src_st, dst_st, gidx, sidx, rows,
                   acc_sh, sem_sd, sem_g, sem_a):
        sc = lax.axis_index("c")
        tid = lax.axis_index("s")
        ebase0 = (sc * (_E // 2) if P == 2 else 0) + tid * ept
        part = sc if P == 2 else 0

        def fire_sd(s, b):
            e0 = ebase0 + b * _B
            pltpu.make_async_copy(srcv.at[pl.ds(e0, _B)], src_st[s],
                                  sem_sd[s]).start()
            pltpu.make_async_copy(dstv.at[pl.ds(e0, _B)], dst_st[s],
                                  sem_sd[s]).start()

        def wait_sd(s):
            pltpu.make_async_copy(srcv.at[pl.ds(0, _B)], src_st[s],
                                  sem_sd[s]).wait()
            pltpu.make_async_copy(dstv.at[pl.ds(0, _B)], dst_st[s],
                                  sem_sd[s]).wait()

        def idx_compute(s, chunk):
            for q in range(_B // 16):
                slq = pl.ds(q * 16, 16)
                if S > 1:
                    gidx[s][slq] = src_st[s][slq] * S + chunk
                else:
                    gidx[s][slq] = src_st[s][slq]
                sidx[s][q // (_IC // 16)][
                    pl.ds((q % (_IC // 16)) * 16, 16)] = dst_st[s][slq]

        def fire_gathers(s):
            for k in range(_NC):
                pltpu.make_async_copy(
                    x128.at[gidx[s].at[pl.ds(k * _IC, _IC)]],
                    rows[s].at[pl.ds(k * _IC, _IC), :], sem_g[s]).start()

        def wait_gathers(s):
            pltpu.make_async_copy(x128.at[pl.ds(0, _B), :], rows[s],
                                  sem_g[s]).wait()

        def fire_scatter(s):
            for k in range(_NC):
                pltpu.async_copy(rows[s].at[pl.ds(k * _IC, _IC), :],
                                 acc_sh.at[sidx[s][k]], sem_a[s], add=True)

        def wait_scatter(s):
            pltpu.make_async_copy(x128.at[pl.ds(0, _B), :], rows[s],
                                  sem_a[s]).wait()

        def step(b, j, chunk):
            o = 1 - j
            wait_gathers(j)
            fire_scatter(j)

            def stage_next():
                wait_sd(o)
                idx_compute(o, chunk)

                def drain_prev():
                    wait_scatter(o)
                _maybe_when(b >= 1, drain_prev)
                fire_gathers(o)
            _maybe_when(b + 1 < nb, stage_next)

            def prefetch():
                fire_sd(j, b + 2)
            _maybe_when(b + 2 < nb, prefetch)

        zrows = rows[0]

        def zero_pass():
            z16 = jnp.zeros((16,), jnp.float32)

            def zb(i, c):
                zrows[i // (_D // 16), pl.ds((i % (_D // 16)) * 16, 16)] = z16
                return c
            lax.fori_loop(0, _B * (_D // 16), zb, 0, unroll=8)
            base = tid * npt
            for off in range(0, npt, _B):
                w = min(_B, npt - off)
                pltpu.sync_copy(zrows.at[pl.ds(0, w), :],
                                acc_sh.at[pl.ds(base + off, w)])

        for p in range(npass):
            chunk = sc * npass + p if S > 1 else 0

            zero_pass()
            plsc.subcore_barrier()

            fire_sd(0, jnp.int32(0))
            fire_sd(1, jnp.int32(1))
            wait_sd(0)
            idx_compute(0, chunk)
            fire_gathers(0)

            def group(g, carry):
                step(2 * g, 0, chunk)
                step(2 * g + 1, 1, chunk)
                return carry
            lax.fori_loop(0, nb // 2, group, 0)
            if nb % 2:
                step(nb - 1, (nb - 1) % 2, chunk)

            wait_scatter(0)
            wait_scatter(1)
            plsc.subcore_barrier()

            base = tid * npt
            pltpu.sync_copy(
                acc_sh.at[pl.ds(base, npt)],
                out.at[part, pl.ds(base, npt), pl.ds(chunk * _D, _D)])
            plsc.subcore_barrier()

    return agg_kernel


_agg_kernels = {F: _make_agg(F) for F in (128, 256, 512)}


def _agg(h, src, dst):
    n, f = h.shape
    parts = _agg_kernels[f](h.reshape(-1, _D), src, dst)
    return parts


def _dense_body(nparts, parts_ref, x_ref, wr_ref, wx_ref, b_ref, o_ref):
    agg = parts_ref[0]
    for p in range(1, nparts):
        agg = agg + parts_ref[p]
    acc = jnp.dot(agg, wr_ref[...], preferred_element_type=jnp.float32)
    acc = acc + jnp.dot(x_ref[...], wx_ref[...], preferred_element_type=jnp.float32)
    o_ref[...] = jnp.maximum(acc + b_ref[...], 0.0)


def _dense(parts, x, w_rel, b, w_root):
    n, f = x.shape
    nparts = parts.shape[0]
    o = w_rel.shape[1]
    bn = 2000
    return pl.pallas_call(
        functools.partial(_dense_body, nparts),
        grid=(n // bn,),
        in_specs=[
            pl.BlockSpec((nparts, bn, f), lambda i: (0, i, 0)),
            pl.BlockSpec((bn, f), lambda i: (i, 0)),
            pl.BlockSpec((f, o), lambda i: (0, 0)),
            pl.BlockSpec((f, o), lambda i: (0, 0)),
            pl.BlockSpec((1, o), lambda i: (0, 0)),
        ],
        out_specs=pl.BlockSpec((bn, o), lambda i: (i, 0)),
        out_shape=jax.ShapeDtypeStruct((n, o), jnp.float32),
    )(parts, x, w_rel, w_root, b.reshape(1, -1))


def _head_body(parts_ref, x_ref, wr_ref, wx_ref, b_ref,
               batch_ref, wm1_ref, bm1_ref, wm2_ref, bm2_ref,
               wm3_ref, bm3_ref, o_ref, pooled_ref, cnt_ref):
    i = pl.program_id(0)
    nsteps = pl.num_programs(0)

    @pl.when(i == 0)
    def _init():
        pooled_ref[...] = jnp.zeros_like(pooled_ref)
        cnt_ref[...] = jnp.zeros_like(cnt_ref)

    acc = jnp.dot(parts_ref[0], wr_ref[...], preferred_element_type=jnp.float32)
    acc = acc + jnp.dot(x_ref[...], wx_ref[...],
                        preferred_element_type=jnp.float32)
    hblk = jnp.maximum(acc + b_ref[...], 0.0)

    bids = batch_ref[0, 0, :]
    gids = jax.lax.broadcasted_iota(jnp.int32, (_G, bids.shape[0]), 0)
    onehot = (gids == bids[None, :]).astype(jnp.float32)
    pooled_ref[...] += jnp.dot(onehot, hblk,
                               preferred_element_type=jnp.float32)
    cnt_ref[...] += jnp.sum(onehot, axis=1, keepdims=True)

    @pl.when(i == nsteps - 1)
    def _final():
        cnt = jnp.maximum(cnt_ref[...], 1.0)
        h = pooled_ref[...] / cnt
        h = jnp.maximum(jnp.dot(h, wm1_ref[...],
                                preferred_element_type=jnp.float32)
                        + bm1_ref[...], 0.0)
        h = jnp.maximum(jnp.dot(h, wm2_ref[...],
                                preferred_element_type=jnp.float32)
                        + bm2_ref[...], 0.0)
        logits = jnp.dot(h, wm3_ref[...],
                         preferred_element_type=jnp.float32) + bm3_ref[...]
        m = jnp.max(logits, axis=-1, keepdims=True)
        z = logits - m
        lse = jnp.log(jnp.sum(jnp.exp(z), axis=-1, keepdims=True))
        o_ref[...] = z - lse


def _head(parts, x, w_rel, b, w_root, batch, wm1, bm1, wm2, bm2, wm3, bm3):
    n, f = x.shape
    o = w_rel.shape[1]
    bn = 2000
    nsteps = n // bn
    batch3 = batch.reshape(nsteps, 1, bn)
    c = wm3.shape[1]
    h1 = wm1.shape[1]
    h2 = wm2.shape[1]
    return pl.pallas_call(
        _head_body,
        grid=(nsteps,),
        in_specs=[
            pl.BlockSpec((1, bn, f), lambda i: (0, i, 0)),
            pl.BlockSpec((bn, f), lambda i: (i, 0)),
            pl.BlockSpec((f, o), lambda i: (0, 0)),
            pl.BlockSpec((f, o), lambda i: (0, 0)),
            pl.BlockSpec((1, o), lambda i: (0, 0)),
            pl.BlockSpec((1, 1, bn), lambda i: (i, 0, 0)),
            pl.BlockSpec((o, h1), lambda i: (0, 0)),
            pl.BlockSpec((1, h1), lambda i: (0, 0)),
            pl.BlockSpec((h1, h2), lambda i: (0, 0)),
            pl.BlockSpec((1, h2), lambda i: (0, 0)),
            pl.BlockSpec((h2, c), lambda i: (0, 0)),
            pl.BlockSpec((1, c), lambda i: (0, 0)),
        ],
        out_specs=pl.BlockSpec((_G, c), lambda i: (0, 0)),
        out_shape=jax.ShapeDtypeStruct((_G, c), jnp.float32),
        scratch_shapes=[
            pltpu.VMEM((_G, o), jnp.float32),
            pltpu.VMEM((_G, 1), jnp.float32),
        ],
    )(parts, x, w_rel, w_root, b.reshape(1, -1), batch3,
      wm1, bm1.reshape(1, -1), wm2, bm2.reshape(1, -1),
      wm3, bm3.reshape(1, -1))


def kernel(x, edge_index, batch,
           W1_rel, b1, W1_root,
           W2_rel, b2, W2_root,
           W3_rel, b3, W3_root,
           Wm1, bm1, Wm2, bm2, Wm3, bm3):
    src = edge_index[0]
    dst = edge_index[1]

    h = _dense(_agg(x, src, dst), x, W1_rel, b1, W1_root)
    h = _dense(_agg(h, src, dst), h, W2_rel, b2, W2_root)
    return _head(_agg(h, src, dst), h, W3_rel, b3, W3_root, batch,
                 Wm1, bm1, Wm2, bm2, Wm3, bm3)
